# Initial kernel scaffold; baseline (speedup 1.0000x reference)
import jax
import jax.numpy as jnp
from jax.experimental import pallas as pl
from jax.experimental.pallas import tpu as pltpu

B, N, M, DIM = 64, 512, 512, 8
INV = 1.0 / (N - 1)
BWD = 0.1
KB = 4
BB = 8


def _sq_kernel(x_ref, y_ref, d_ref):
    for bb in range(x_ref.shape[0]):
        x = x_ref[bb]
        y = y_ref[bb]
        x2 = jnp.sum(x * x, axis=1, keepdims=True)
        y2 = jnp.sum(y * y, axis=1, keepdims=True)
        one = jnp.ones_like(x2)
        a = jnp.concatenate([x * -2.0, x2, one], axis=1)
        b = jnp.concatenate([y, one, y2], axis=1)
        d_ref[bb] = jax.lax.dot_general(
            a, b, (((1,), (1,)), ((), ())),
            preferred_element_type=jnp.float32)


def _shift_r(x, s, fill):
    bb = x.shape[0]
    return jnp.concatenate(
        [jnp.full((bb, s), fill, jnp.float32), x[:, :-s]], axis=1)


def _dp_kernel(d_ref, r_ref):
    bb = d_ref.shape[1]
    inf = jnp.float32(jnp.inf)
    jf = jax.lax.broadcasted_iota(jnp.float32, (bb, N), 1)

    def row_step(orow, prev):
        i = orow - 1
        drow = d_ref[jnp.clip(i, 0, N - 1)]
        pad = jnp.where(i == 0, jnp.float32(0.0), inf)
        prevm1 = jnp.concatenate(
            [jnp.broadcast_to(pad, (bb, 1)), prev[:, :-1]], axis=1)
        a = jnp.minimum(prevm1, prev)
        c = drow
        for s in (1, 2, 4, 8, 16, 32, 64, 128, 256):
            c = c + _shift_r(c, s, 0.0)
        cume = c - drow
        iif = i.astype(jnp.float32)
        band = jnp.abs(iif * jnp.float32(INV) - jf * jnp.float32(INV)) \
            <= jnp.float32(BWD)
        valid = jnp.logical_and(band,
                                jnp.logical_and(i >= 0, i < N))
        p = jnp.where(valid, a - cume, inf)
        m = p
        for s in (1, 2, 4, 8, 16, 32, 64):
            m = jnp.minimum(m, _shift_r(m, s, inf))
        r = jnp.where(valid, c + m, inf)
        lp = jnp.where(orow == 0, jnp.float32(0.0), inf)
        tile = jnp.concatenate(
            [jnp.broadcast_to(lp, (1, bb, 1)), r[None],
             jnp.full((1, bb, 1), inf, jnp.float32)], axis=2)
        r_ref[pl.ds(orow, 1)] = tile
        return r

    init = jnp.full((bb, N), inf, jnp.float32)
    jax.lax.fori_loop(0, N + 2, row_step, init)


def kernel(X, Y):
    D = pl.pallas_call(
        _sq_kernel,
        grid=(B // KB,),
        in_specs=[
            pl.BlockSpec((KB, N, DIM), lambda b: (b, 0, 0)),
            pl.BlockSpec((KB, M, DIM), lambda b: (b, 0, 0)),
        ],
        out_specs=pl.BlockSpec((KB, N, M), lambda b: (b, 0, 0)),
        out_shape=jax.ShapeDtypeStruct((B, N, M), jnp.float32),
        compiler_params=pltpu.CompilerParams(
            dimension_semantics=("arbitrary",),
            vmem_limit_bytes=48 * 1024 * 1024,
        ),
        name="sqdist",
    )(X, Y)
    Dt = jnp.transpose(D, (1, 0, 2))
    Rtmp = pl.pallas_call(
        _dp_kernel,
        grid=(B // BB,),
        in_specs=[pl.BlockSpec((N, BB, M), lambda g: (0, g, 0))],
        out_specs=pl.BlockSpec((N + 2, BB, M + 2), lambda g: (0, g, 0)),
        out_shape=jax.ShapeDtypeStruct((N + 2, B, M + 2), jnp.float32),
        compiler_params=pltpu.CompilerParams(
            dimension_semantics=("arbitrary",),
            vmem_limit_bytes=52 * 1024 * 1024,
        ),
        name="dtw_dp",
    )(Dt)
    R = jnp.transpose(Rtmp, (1, 0, 2))
    loss = R[:, N, M]
    return (loss, R)


# speedup vs baseline: 2.5148x; 2.5148x over previous
import jax
import jax.numpy as jnp
from jax.experimental import pallas as pl
from jax.experimental.pallas import tpu as pltpu

B, N, M, DIM = 64, 512, 512, 8
INV = 1.0 / (N - 1)
BWD = 0.1
KB = 4
BB = 8


def _sq_kernel(x_ref, y_ref, d_ref):
    for bb in range(x_ref.shape[0]):
        x = x_ref[bb]
        y = y_ref[bb]
        x2 = jnp.sum(x * x, axis=1, keepdims=True)
        y2 = jnp.sum(y * y, axis=1, keepdims=True)
        one = jnp.ones_like(x2)
        a = jnp.concatenate([x * -2.0, x2, one], axis=1)
        b = jnp.concatenate([y, one, y2], axis=1)
        d_ref[bb] = jax.lax.dot_general(
            a, b, (((1,), (1,)), ((), ())),
            preferred_element_type=jnp.float32)


def _shift_r(x, s, fill):
    bb = x.shape[0]
    return jnp.concatenate(
        [jnp.full((bb, s), fill, jnp.float32), x[:, :-s]], axis=1)


def _dp_kernel(d_ref, r_ref):
    bb = d_ref.shape[1]
    inf = jnp.float32(jnp.inf)
    jf = jax.lax.broadcasted_iota(jnp.int32, (bb, N), 1).astype(jnp.float32)

    def row_step(orow, prev):
        i = orow - 1
        drow = d_ref[jnp.clip(i, 0, N - 1)]
        pad = jnp.where(i == 0, jnp.float32(0.0), inf)
        prevm1 = jnp.concatenate(
            [jnp.broadcast_to(pad, (bb, 1)), prev[:, :-1]], axis=1)
        a = jnp.minimum(prevm1, prev)
        c = drow
        for s in (1, 2, 4, 8, 16, 32, 64, 128, 256):
            c = c + _shift_r(c, s, 0.0)
        cume = c - drow
        iif = i.astype(jnp.float32)
        band = jnp.abs(iif * jnp.float32(INV) - jf * jnp.float32(INV)) \
            <= jnp.float32(BWD)
        valid = jnp.logical_and(band,
                                jnp.logical_and(i >= 0, i < N))
        p = jnp.where(valid, a - cume, inf)
        m = p
        for s in (1, 2, 4, 8, 16, 32, 64):
            m = jnp.minimum(m, _shift_r(m, s, inf))
        r = jnp.where(valid, c + m, inf)
        lp = jnp.where(orow == 0, jnp.float32(0.0), inf)
        tile = jnp.concatenate(
            [jnp.broadcast_to(lp, (1, bb, 1)), r[None],
             jnp.full((1, bb, 1), inf, jnp.float32)], axis=2)
        r_ref[pl.ds(orow, 1)] = tile
        return r

    init = jnp.full((bb, N), inf, jnp.float32)
    jax.lax.fori_loop(0, N + 2, row_step, init)


def kernel(X, Y):
    D = pl.pallas_call(
        _sq_kernel,
        grid=(B // KB,),
        in_specs=[
            pl.BlockSpec((KB, N, DIM), lambda b: (b, 0, 0)),
            pl.BlockSpec((KB, M, DIM), lambda b: (b, 0, 0)),
        ],
        out_specs=pl.BlockSpec((KB, N, M), lambda b: (b, 0, 0)),
        out_shape=jax.ShapeDtypeStruct((B, N, M), jnp.float32),
        compiler_params=pltpu.CompilerParams(
            dimension_semantics=("arbitrary",),
            vmem_limit_bytes=48 * 1024 * 1024,
        ),
        name="sqdist",
    )(X, Y)
    Dt = jnp.transpose(D, (1, 0, 2))
    Rtmp = pl.pallas_call(
        _dp_kernel,
        grid=(B // BB,),
        in_specs=[pl.BlockSpec((N, BB, M), lambda g: (0, g, 0))],
        out_specs=pl.BlockSpec((N + 2, BB, M + 2), lambda g: (0, g, 0)),
        out_shape=jax.ShapeDtypeStruct((N + 2, B, M + 2), jnp.float32),
        compiler_params=pltpu.CompilerParams(
            dimension_semantics=("arbitrary",),
            vmem_limit_bytes=52 * 1024 * 1024,
        ),
        name="dtw_dp",
    )(Dt)
    R = jnp.transpose(Rtmp, (1, 0, 2))
    loss = R[:, N, M]
    return (loss, R)


# speedup vs baseline: 13.5907x; 5.4043x over previous
import jax
import jax.numpy as jnp
from jax.experimental import pallas as pl
from jax.experimental.pallas import tpu as pltpu

B, N, M, DIM = 64, 512, 512, 8
INV = 1.0 / (N - 1)
BWD = 0.1
KB = 4
RC = 32
CH = 17


def _sq_kernel(x_ref, y_ref, d_ref):
    for bb in range(x_ref.shape[0]):
        x = x_ref[bb]
        y = y_ref[bb]
        x2 = jnp.sum(x * x, axis=1, keepdims=True)
        y2 = jnp.sum(y * y, axis=1, keepdims=True)
        one = jnp.ones_like(x2)
        a = jnp.concatenate([x * -2.0, x2, one], axis=1)
        b = jnp.concatenate([y, one, y2], axis=1)
        d_ref[bb] = jax.lax.dot_general(
            a, b, (((1,), (1,)), ((), ())),
            preferred_element_type=jnp.float32)


def _shift_r(x, s, fill):
    bb = x.shape[0]
    return jnp.concatenate(
        [jnp.full((bb, s), fill, jnp.float32), x[:, :-s]], axis=1)


def _dp_kernel(d_ref, r_ref, prev_sc):
    bb = d_ref.shape[1]
    rows = d_ref.shape[0]
    inf = jnp.float32(jnp.inf)
    jf = jax.lax.broadcasted_iota(jnp.int32, (bb, N), 1).astype(jnp.float32)
    c_id = pl.program_id(0)

    @pl.when(c_id == 0)
    def _():
        prev_sc[...] = jnp.full((bb, N), inf, jnp.float32)

    def row_step(r, prev):
        orow = c_id * rows + r
        i = orow - 1
        drow = d_ref[r]
        pad = jnp.where(i == 0, jnp.float32(0.0), inf)
        prevm1 = jnp.concatenate(
            [jnp.broadcast_to(pad, (bb, 1)), prev[:, :-1]], axis=1)
        a = jnp.minimum(prevm1, prev)
        c = drow
        for s in (1, 2, 4, 8, 16, 32, 64, 128, 256):
            c = c + _shift_r(c, s, 0.0)
        cume = c - drow
        iif = i.astype(jnp.float32)
        band = jnp.abs(iif * jnp.float32(INV) - jf * jnp.float32(INV)) \
            <= jnp.float32(BWD)
        valid = jnp.logical_and(band,
                                jnp.logical_and(i >= 0, i < N))
        p = jnp.where(valid, a - cume, inf)
        m = p
        for s in (1, 2, 4, 8, 16, 32, 64):
            m = jnp.minimum(m, _shift_r(m, s, inf))
        rr = jnp.where(valid, c + m, inf)
        lp = jnp.where(orow == 0, jnp.float32(0.0), inf)
        tile = jnp.concatenate(
            [jnp.broadcast_to(lp, (1, bb, 1)), rr[None],
             jnp.full((1, bb, 1), inf, jnp.float32)], axis=2)
        r_ref[pl.ds(r, 1)] = tile
        return rr

    prev = jax.lax.fori_loop(0, rows, row_step, prev_sc[...])
    prev_sc[...] = prev


def kernel(X, Y):
    D = pl.pallas_call(
        _sq_kernel,
        grid=(B // KB,),
        in_specs=[
            pl.BlockSpec((KB, N, DIM), lambda b: (b, 0, 0)),
            pl.BlockSpec((KB, M, DIM), lambda b: (b, 0, 0)),
        ],
        out_specs=pl.BlockSpec((KB, N, M), lambda b: (b, 0, 0)),
        out_shape=jax.ShapeDtypeStruct((B, N, M), jnp.float32),
        compiler_params=pltpu.CompilerParams(
            dimension_semantics=("arbitrary",),
            vmem_limit_bytes=48 * 1024 * 1024,
        ),
        name="sqdist",
    )(X, Y)
    Dpad = jnp.pad(jnp.transpose(D, (1, 0, 2)), ((1, CH * RC - N - 1), (0, 0), (0, 0)))
    Rtmp = pl.pallas_call(
        _dp_kernel,
        grid=(CH,),
        in_specs=[pl.BlockSpec((RC, B, M), lambda c: (c, 0, 0))],
        out_specs=pl.BlockSpec((RC, B, M + 2), lambda c: (c, 0, 0)),
        out_shape=jax.ShapeDtypeStruct((CH * RC, B, M + 2), jnp.float32),
        scratch_shapes=[pltpu.VMEM((B, N), jnp.float32)],
        compiler_params=pltpu.CompilerParams(
            dimension_semantics=("arbitrary",),
            vmem_limit_bytes=52 * 1024 * 1024,
        ),
        name="dtw_dp",
    )(Dpad)
    R = jnp.transpose(Rtmp[:N + 2], (1, 0, 2))
    loss = R[:, N, M]
    return (loss, R)


# speedup vs baseline: 17.1251x; 1.2601x over previous
import jax
import jax.numpy as jnp
from jax.experimental import pallas as pl
from jax.experimental.pallas import tpu as pltpu

B, N, M, DIM = 64, 512, 512, 8
INV = 1.0 / (N - 1)
BWD = 0.1
KB = 4
RC = 32
CH = 17


def _sq_kernel(x_ref, y_ref, d_ref):
    for bb in range(x_ref.shape[0]):
        x = x_ref[bb]
        y = y_ref[bb]
        x2 = jnp.sum(x * x, axis=1, keepdims=True)
        y2 = jnp.sum(y * y, axis=1, keepdims=True)
        one = jnp.ones_like(x2)
        a = jnp.concatenate([x * -2.0, x2, one], axis=1)
        b = jnp.concatenate([y, one, y2], axis=1)
        d_ref[bb] = jax.lax.dot_general(
            a, b, (((1,), (1,)), ((), ())),
            preferred_element_type=jnp.float32)


def _shift_r(x, s, fill):
    bb = x.shape[0]
    return jnp.concatenate(
        [jnp.full((bb, s), fill, jnp.float32), x[:, :-s]], axis=1)


def _dp_kernel(d_ref, r_ref, prev_sc):
    rows = d_ref.shape[0]
    inf = jnp.float32(jnp.inf)
    ji = jax.lax.broadcasted_iota(jnp.int32, (B, N), 1)
    c_id = pl.program_id(0)

    @pl.when(c_id == 0)
    def _():
        prev_sc[...] = jnp.full((B, N), inf, jnp.float32)

    prev = prev_sc[...]
    for r in range(rows):
        orow = c_id * rows + r
        i = orow - 1
        drow = d_ref[r]
        pad = jnp.where(i == 0, jnp.float32(0.0), inf)
        prevm1 = jnp.concatenate(
            [jnp.broadcast_to(pad, (B, 1)), prev[:, :-1]], axis=1)
        a = jnp.minimum(prevm1, prev)
        valid = jnp.logical_and(
            jnp.logical_and(ji >= i - 51, ji <= i + 51),
            jnp.logical_and(i >= 0, i < N))
        mm = jnp.where(valid, a + drow, inf)
        ms = drow
        for shifts in ((1, 2, 3), (4, 8, 12), (16, 32, 48), (64,)):
            acc = ms
            new_mm = mm
            for t, s in enumerate(shifts):
                new_mm = jnp.minimum(new_mm, _shift_r(mm, s, inf) + acc)
                if t + 1 < len(shifts):
                    acc = acc + _shift_r(ms, s, inf)
            if shifts[0] != 64:
                ms = acc + _shift_r(ms, shifts[-1], inf)
            mm = new_mm
        rr = jnp.where(valid, mm, inf)
        lp = jnp.where(orow == 0, jnp.float32(0.0), inf)
        tile = jnp.concatenate(
            [jnp.broadcast_to(lp, (1, B, 1)), rr[None],
             jnp.full((1, B, 1), inf, jnp.float32)], axis=2)
        r_ref[r] = tile[0]
        prev = rr
    prev_sc[...] = prev


def kernel(X, Y):
    D = pl.pallas_call(
        _sq_kernel,
        grid=(B // KB,),
        in_specs=[
            pl.BlockSpec((KB, N, DIM), lambda b: (b, 0, 0)),
            pl.BlockSpec((KB, M, DIM), lambda b: (b, 0, 0)),
        ],
        out_specs=pl.BlockSpec((KB, N, M), lambda b: (b, 0, 0)),
        out_shape=jax.ShapeDtypeStruct((B, N, M), jnp.float32),
        compiler_params=pltpu.CompilerParams(
            dimension_semantics=("arbitrary",),
            vmem_limit_bytes=48 * 1024 * 1024,
        ),
        name="sqdist",
    )(X, Y)
    Dpad = jnp.pad(jnp.transpose(D, (1, 0, 2)), ((1, CH * RC - N - 1), (0, 0), (0, 0)))
    Rtmp = pl.pallas_call(
        _dp_kernel,
        grid=(CH,),
        in_specs=[pl.BlockSpec((RC, B, M), lambda c: (c, 0, 0))],
        out_specs=pl.BlockSpec((RC, B, M + 2), lambda c: (c, 0, 0)),
        out_shape=jax.ShapeDtypeStruct((CH * RC, B, M + 2), jnp.float32),
        scratch_shapes=[pltpu.VMEM((B, N), jnp.float32)],
        compiler_params=pltpu.CompilerParams(
            dimension_semantics=("arbitrary",),
            vmem_limit_bytes=52 * 1024 * 1024,
        ),
        name="dtw_dp",
    )(Dpad)
    R = jnp.transpose(Rtmp[:N + 2], (1, 0, 2))
    loss = R[:, N, M]
    return (loss, R)


# speedup vs baseline: 19.3835x; 1.1319x over previous
import jax
import jax.numpy as jnp
from jax.experimental import pallas as pl
from jax.experimental.pallas import tpu as pltpu

B, N, M, DIM = 64, 512, 512, 8
INV = 1.0 / (N - 1)
BWD = 0.1
KB = 4
RC = 32
CH = 17


def _sq_kernel(x_ref, y_ref, d_ref):
    for bb in range(x_ref.shape[0]):
        x = x_ref[bb]
        y = y_ref[bb]
        x2 = jnp.sum(x * x, axis=1, keepdims=True)
        y2 = jnp.sum(y * y, axis=1, keepdims=True)
        one = jnp.ones_like(x2)
        a = jnp.concatenate([x * -2.0, x2, one], axis=1)
        b = jnp.concatenate([y, one, y2], axis=1)
        d_ref[bb] = jax.lax.dot_general(
            a, b, (((1,), (1,)), ((), ())),
            preferred_element_type=jnp.float32).astype(jnp.bfloat16)


def _shift_r(x, s, fill):
    bb = x.shape[0]
    return jnp.concatenate(
        [jnp.full((bb, s), fill, jnp.float32), x[:, :-s]], axis=1)


def _dp_kernel(d_ref, r_ref, prev_sc):
    rows = d_ref.shape[0]
    bsz = d_ref.shape[1]
    inf = jnp.float32(jnp.inf)
    ji = jax.lax.broadcasted_iota(jnp.int32, (bsz, N), 1)
    c_id = pl.program_id(0)

    @pl.when(c_id == 0)
    def _():
        prev_sc[...] = jnp.full((bsz, N), inf, jnp.float32)

    prev = prev_sc[...]
    for r in range(rows):
        orow = c_id * rows + r
        i = orow - 1
        drow = d_ref[r].astype(jnp.float32)
        pad = jnp.where(i == 0, jnp.float32(0.0), inf)
        prevm1 = jnp.concatenate(
            [jnp.broadcast_to(pad, (bsz, 1)), prev[:, :-1]], axis=1)
        a = jnp.minimum(prevm1, prev)
        valid = jnp.logical_and(
            jnp.logical_and(ji >= i - 51, ji <= i + 51),
            jnp.logical_and(i >= 0, i < N))
        mm = jnp.where(valid, a + drow, inf)
        ms = drow
        for shifts in ((1, 2, 3), (4, 8, 12), (16, 32, 48), (64,)):
            acc = ms
            new_mm = mm
            for t, s in enumerate(shifts):
                new_mm = jnp.minimum(new_mm, _shift_r(mm, s, inf) + acc)
                if t + 1 < len(shifts):
                    acc = acc + _shift_r(ms, s, inf)
            if shifts[0] != 64:
                ms = acc + _shift_r(ms, shifts[-1], inf)
            mm = new_mm
        rr = jnp.where(valid, mm, inf)
        lp = jnp.where(orow == 0, jnp.float32(0.0), inf)
        tile = jnp.concatenate(
            [jnp.broadcast_to(lp, (1, bsz, 1)), rr[None],
             jnp.full((1, bsz, 1), inf, jnp.float32)], axis=2)
        r_ref[r] = tile[0]
        prev = rr
    prev_sc[...] = prev


def kernel(X, Y):
    D = pl.pallas_call(
        _sq_kernel,
        grid=(B // KB,),
        in_specs=[
            pl.BlockSpec((KB, N, DIM), lambda b: (b, 0, 0)),
            pl.BlockSpec((KB, M, DIM), lambda b: (b, 0, 0)),
        ],
        out_specs=pl.BlockSpec((KB, N, M), lambda b: (b, 0, 0)),
        out_shape=jax.ShapeDtypeStruct((B, N, M), jnp.bfloat16),
        compiler_params=pltpu.CompilerParams(
            dimension_semantics=("arbitrary",),
            vmem_limit_bytes=48 * 1024 * 1024,
        ),
        name="sqdist",
    )(X, Y)
    Dpad = jnp.pad(jnp.transpose(D, (1, 0, 2)), ((1, CH * RC - N - 1), (0, 0), (0, 0)))
    Rtmp = pl.pallas_call(
        _dp_kernel,
        grid=(CH,),
        in_specs=[pl.BlockSpec((RC, B, M), lambda c: (c, 0, 0))],
        out_specs=pl.BlockSpec((RC, B, M + 2), lambda c: (c, 0, 0)),
        out_shape=jax.ShapeDtypeStruct((CH * RC, B, M + 2), jnp.float32),
        scratch_shapes=[pltpu.VMEM((B, N), jnp.float32)],
        compiler_params=pltpu.CompilerParams(
            dimension_semantics=("arbitrary",),
            vmem_limit_bytes=52 * 1024 * 1024,
        ),
        name="dtw_dp",
    )(Dpad)
    R = jnp.transpose(Rtmp[:N + 2], (1, 0, 2))
    loss = R[:, N, M]
    return (loss, R)


# speedup vs baseline: 24.3291x; 1.2551x over previous
import jax
import jax.numpy as jnp
from jax.experimental import pallas as pl
from jax.experimental.pallas import tpu as pltpu

B, N, M, DIM = 64, 512, 512, 8
INV = 1.0 / (N - 1)
BWD = 0.1
KB = 4
RC = 32
CH = 17


def _sq_kernel(x_ref, y_ref, d_ref):
    for bb in range(x_ref.shape[0]):
        x = x_ref[bb]
        y = y_ref[bb]
        x2 = jnp.sum(x * x, axis=1, keepdims=True)
        y2 = jnp.sum(y * y, axis=1, keepdims=True)
        one = jnp.ones_like(x2)
        a = jnp.concatenate([x * -2.0, x2, one], axis=1)
        b = jnp.concatenate([y, one, y2], axis=1)
        d_ref[bb] = jax.lax.dot_general(
            a, b, (((1,), (1,)), ((), ())),
            preferred_element_type=jnp.float32).astype(jnp.bfloat16)


def _shift_r(x, s, fill):
    bb = x.shape[0]
    return jnp.concatenate(
        [jnp.full((bb, s), fill, jnp.float32), x[:, :-s]], axis=1)


W = 128


def _dp_kernel(d_ref, r_ref, prev_sc):
    rows = d_ref.shape[0]
    bsz = d_ref.shape[1]
    inf = jnp.float32(jnp.inf)
    wi = jax.lax.broadcasted_iota(jnp.int32, (bsz, W), 1)
    c_id = pl.program_id(0)

    @pl.when(c_id == 0)
    def _():
        prev_sc[...] = jnp.full((bsz, W), inf, jnp.float32)

    prev = prev_sc[...]
    for r in range(rows):
        orow = c_id * rows + r
        i = orow - 1
        o = jnp.clip(orow - 52, 0, N - W)
        op = jnp.clip(orow - 53, 0, N - W)
        delta = o - op
        s0 = jnp.minimum((o >> 7) << 7, N - 2 * W)
        slab = d_ref[r, :, pl.ds(pl.multiple_of(s0, W), 2 * W)]
        slab = pltpu.roll(slab.astype(jnp.float32), s0 - o + 2 * W, axis=1)
        dw = slab[:, :W]
        jw = wi + o
        prev_c = pltpu.roll(prev, W - delta, axis=1)
        prevm1 = pltpu.roll(prev, 1 - delta, axis=1)
        pad = jnp.where(i == 0, jnp.float32(0.0), inf)
        prevm1 = jnp.where(jw == 0, pad, prevm1)
        a = jnp.minimum(prevm1, prev_c)
        valid = jnp.logical_and(
            jnp.logical_and(jw >= i - 51, jw <= i + 51),
            jnp.logical_and(i >= 0, i < N))
        mm = jnp.where(valid, a + dw, inf)
        ms = dw
        for shifts in ((1, 2, 3), (4, 8, 12), (16, 32, 48), (64,)):
            acc = ms
            new_mm = mm
            for t, s in enumerate(shifts):
                new_mm = jnp.minimum(new_mm, _shift_r(mm, s, inf) + acc)
                if t + 1 < len(shifts):
                    acc = acc + _shift_r(ms, s, inf)
            if shifts[0] != 64:
                ms = acc + _shift_r(ms, shifts[-1], inf)
            mm = new_mm
        rr = jnp.where(valid, mm, inf)
        full = jnp.concatenate(
            [rr, jnp.full((bsz, M - W), inf, jnp.float32)], axis=1)
        full = pltpu.roll(full, o, axis=1)
        lp = jnp.where(orow == 0, jnp.float32(0.0), inf)
        tile = jnp.concatenate(
            [jnp.broadcast_to(lp, (bsz, 1)), full,
             jnp.full((bsz, 1), inf, jnp.float32)], axis=1)
        r_ref[r] = tile
        prev = rr
    prev_sc[...] = prev


def kernel(X, Y):
    D = pl.pallas_call(
        _sq_kernel,
        grid=(B // KB,),
        in_specs=[
            pl.BlockSpec((KB, N, DIM), lambda b: (b, 0, 0)),
            pl.BlockSpec((KB, M, DIM), lambda b: (b, 0, 0)),
        ],
        out_specs=pl.BlockSpec((KB, N, M), lambda b: (b, 0, 0)),
        out_shape=jax.ShapeDtypeStruct((B, N, M), jnp.bfloat16),
        compiler_params=pltpu.CompilerParams(
            dimension_semantics=("arbitrary",),
            vmem_limit_bytes=48 * 1024 * 1024,
        ),
        name="sqdist",
    )(X, Y)
    Dpad = jnp.pad(jnp.transpose(D, (1, 0, 2)), ((1, CH * RC - N - 1), (0, 0), (0, 0)))
    Rtmp = pl.pallas_call(
        _dp_kernel,
        grid=(CH,),
        in_specs=[pl.BlockSpec((RC, B, M), lambda c: (c, 0, 0))],
        out_specs=pl.BlockSpec((RC, B, M + 2), lambda c: (c, 0, 0)),
        out_shape=jax.ShapeDtypeStruct((CH * RC, B, M + 2), jnp.float32),
        scratch_shapes=[pltpu.VMEM((B, W), jnp.float32)],
        compiler_params=pltpu.CompilerParams(
            dimension_semantics=("arbitrary",),
            vmem_limit_bytes=52 * 1024 * 1024,
        ),
        name="dtw_dp",
    )(Dpad)
    R = jnp.transpose(Rtmp[:N + 2], (1, 0, 2))
    loss = R[:, N, M]
    return (loss, R)


# speedup vs baseline: 24.4002x; 1.0029x over previous
import jax
import jax.numpy as jnp
from jax.experimental import pallas as pl
from jax.experimental.pallas import tpu as pltpu

B, N, M, DIM = 64, 512, 512, 8
INV = 1.0 / (N - 1)
BWD = 0.1
KB = 4
RC = 16
CH = 34


def _sq_kernel(x_ref, y_ref, d_ref):
    for bb in range(x_ref.shape[0]):
        x = x_ref[bb]
        y = y_ref[bb]
        x2 = jnp.sum(x * x, axis=1, keepdims=True)
        y2 = jnp.sum(y * y, axis=1, keepdims=True)
        one = jnp.ones_like(x2)
        a = jnp.concatenate([x * -2.0, x2, one], axis=1)
        b = jnp.concatenate([y, one, y2], axis=1)
        d_ref[bb] = jax.lax.dot_general(
            a, b, (((1,), (1,)), ((), ())),
            preferred_element_type=jnp.float32).astype(jnp.bfloat16)


def _shift_r(x, s, fill):
    bb = x.shape[0]
    return jnp.concatenate(
        [jnp.full((bb, s), fill, jnp.float32), x[:, :-s]], axis=1)


GRP = 1
W = 256


def _lb(c):
    return jnp.clip(jnp.floor_divide(16 * c - 52, 128), 0, 2)


def _dp_kernel(d0_ref, d1_ref, r_ref, prev_sc):
    rows = d0_ref.shape[0]
    bsz = d0_ref.shape[1]
    inf = jnp.float32(jnp.inf)
    wi = jax.lax.broadcasted_iota(jnp.int32, (bsz, W), 1)
    c_id = pl.program_id(0)
    lb = _lb(c_id)
    o = lb * 128

    @pl.when(c_id == 0)
    def _():
        prev_sc[...] = jnp.full((bsz, W), inf, jnp.float32)

    dlb = lb - _lb(c_id - 1)
    prev_all = prev_sc[...]
    prev_shift = jnp.concatenate(
        [prev_all[:, 128:], jnp.full((bsz, 128), inf, jnp.float32)], axis=1)
    prev_all = jnp.where(dlb == 1, prev_shift, prev_all)
    jw = wi[:bsz // GRP]
    jw = jw + o

    gb = bsz // GRP
    prev = [prev_all[g * gb:(g + 1) * gb, :] for g in range(GRP)]

    def row_body(r, g):
        orow = c_id * rows + r
        i = orow - 1
        dw = jnp.concatenate(
            [d0_ref[r, g * gb:(g + 1) * gb, :],
             d1_ref[r, g * gb:(g + 1) * gb, :]],
            axis=1).astype(jnp.float32)
        pad = jnp.where(i == 0, jnp.float32(0.0), inf)
        prevm1 = jnp.where(jw == 0, pad, _shift_r(prev[g], 1, jnp.inf))
        a = jnp.minimum(prevm1, prev[g])
        valid = jnp.logical_and(
            jnp.logical_and(jw >= i - 51, jw <= i + 51),
            jnp.logical_and(i >= 0, i < N))
        mm = jnp.where(valid, a + dw, inf)
        ms = dw
        for shifts in ((1, 2, 3), (4, 8, 12), (16, 32, 48), (64,)):
            acc = ms
            new_mm = mm
            for t, s in enumerate(shifts):
                new_mm = jnp.minimum(new_mm, _shift_r(mm, s, inf) + acc)
                if t + 1 < len(shifts):
                    acc = acc + _shift_r(ms, s, inf)
            if shifts[0] != 64:
                ms = acc + _shift_r(ms, shifts[-1], inf)
            mm = new_mm
        rr = jnp.where(valid, mm, inf)
        parts = [jnp.where(lb == t, rr[:, :128],
                           jnp.where(lb == t - 1, rr[:, 128:],
                                     jnp.full((gb, 128), inf, jnp.float32)))
                 for t in range(4)]
        lp = jnp.where(orow == 0, jnp.float32(0.0), inf)
        tile = jnp.concatenate(
            [jnp.broadcast_to(lp, (gb, 1))] + parts +
            [jnp.full((gb, 1), inf, jnp.float32)], axis=1)
        r_ref[r, g * gb:(g + 1) * gb, :] = tile
        prev[g] = rr

    for step in range(rows + GRP - 1):
        for g in range(GRP):
            r = step - g
            if 0 <= r < rows:
                row_body(r, g)
    prev_sc[...] = jnp.concatenate(prev, axis=0)


def kernel(X, Y):
    D = pl.pallas_call(
        _sq_kernel,
        grid=(B // KB,),
        in_specs=[
            pl.BlockSpec((KB, N, DIM), lambda b: (b, 0, 0)),
            pl.BlockSpec((KB, M, DIM), lambda b: (b, 0, 0)),
        ],
        out_specs=pl.BlockSpec((KB, N, M), lambda b: (b, 0, 0)),
        out_shape=jax.ShapeDtypeStruct((B, N, M), jnp.bfloat16),
        compiler_params=pltpu.CompilerParams(
            dimension_semantics=("arbitrary",),
            vmem_limit_bytes=48 * 1024 * 1024,
        ),
        name="sqdist",
    )(X, Y)
    Dpad = jnp.pad(jnp.transpose(D, (1, 0, 2)), ((1, CH * RC - N - 1), (0, 0), (0, 0)))
    Rtmp = pl.pallas_call(
        _dp_kernel,
        grid=(CH,),
        in_specs=[
            pl.BlockSpec((RC, B, 128), lambda c: (c, 0, _lb(c))),
            pl.BlockSpec((RC, B, 128), lambda c: (c, 0, _lb(c) + 1)),
        ],
        out_specs=pl.BlockSpec((RC, B, M + 2), lambda c: (c, 0, 0)),
        out_shape=jax.ShapeDtypeStruct((CH * RC, B, M + 2), jnp.float32),
        scratch_shapes=[pltpu.VMEM((B, W), jnp.float32)],
        compiler_params=pltpu.CompilerParams(
            dimension_semantics=("arbitrary",),
            vmem_limit_bytes=52 * 1024 * 1024,
        ),
        name="dtw_dp",
    )(Dpad, Dpad)
    R = jnp.transpose(Rtmp[:N + 2], (1, 0, 2))
    loss = R[:, N, M]
    return (loss, R)
